# Initial kernel scaffold; baseline (speedup 1.0000x reference)
import functools

import jax
import jax.numpy as jnp
from jax import lax
from jax.experimental import pallas as pl
from jax.experimental.pallas import tpu as pltpu
from jax.experimental.pallas import tpu_sc as plsc

N = 10000
E = 320000
D = 128
NG = 64
NP = 10240
EP = 323584
CH = 128
CPT = EP // (32 * CH)
RPS = NP // 16
NBLK = 16
BR = NP // NBLK

_SC_MESH = plsc.VectorSubcoreMesh(core_axis_name="c", subcore_axis_name="s")


def _deg_body(col2_hbm, ones_hbm, zeros_hbm, out_hbm, acc_sh, idx_v, ones_v):
    c = lax.axis_index("c")
    s = lax.axis_index("s")
    pltpu.sync_copy(zeros_hbm.at[pl.ds(s * RPS, RPS)],
                    acc_sh.at[pl.ds(s * RPS, RPS)])
    pltpu.sync_copy(ones_hbm, ones_v)
    plsc.subcore_barrier()
    base = (c * 16 + s) * CPT

    @pl.loop(0, CPT)
    def _(j):
        pltpu.sync_copy(col2_hbm.at[base + j], idx_v.at[0])
        pltpu.sync_copy(ones_v, acc_sh.at[idx_v.at[0]], add=True)

    plsc.subcore_barrier()
    pltpu.sync_copy(acc_sh.at[pl.ds(s * RPS, RPS)],
                    out_hbm.at[c, pl.ds(s * RPS, RPS)])


@jax.jit
def _sc_degrees(col2, ones16, zeros16):
    return pl.kernel(
        _deg_body,
        out_type=jax.ShapeDtypeStruct((2, NP, 16), jnp.float32),
        mesh=_SC_MESH,
        scratch_types=[
            pltpu.VMEM_SHARED((NP, 16), jnp.float32),
            pltpu.VMEM((1, CH), jnp.int32),
            pltpu.VMEM((CH, 16), jnp.float32),
        ],
    )(col2, ones16, zeros16)


def _agg_body(h_hbm, col2_hbm, row2_hbm, zeros_hbm, out_hbm,
              acc_sh, cidx, ridx, msg, sem):
    c = lax.axis_index("c")
    s = lax.axis_index("s")
    pltpu.sync_copy(zeros_hbm.at[pl.ds(s * RPS, RPS)],
                    acc_sh.at[pl.ds(s * RPS, RPS)])
    plsc.subcore_barrier()
    base = (c * 16 + s) * CPT

    @pl.loop(0, CPT)
    def _(j):
        pltpu.sync_copy(col2_hbm.at[base + j], cidx.at[0])
        pltpu.sync_copy(row2_hbm.at[base + j], ridx.at[0])
        pltpu.async_copy(h_hbm.at[cidx.at[0]], msg, sem).wait()
        pltpu.sync_copy(msg, acc_sh.at[ridx.at[0]], add=True)

    plsc.subcore_barrier()
    pltpu.sync_copy(acc_sh.at[pl.ds(s * RPS, RPS)],
                    out_hbm.at[c, pl.ds(s * RPS, RPS)])


@jax.jit
def _sc_aggregate(h, col2, row2, zeros_nd):
    return pl.kernel(
        _agg_body,
        out_type=jax.ShapeDtypeStruct((2, NP, D), jnp.float32),
        mesh=_SC_MESH,
        scratch_types=[
            pltpu.VMEM_SHARED((NP, D), jnp.float32),
            pltpu.VMEM((1, CH), jnp.int32),
            pltpu.VMEM((1, CH), jnp.int32),
            pltpu.VMEM((CH, D), jnp.float32),
            pltpu.SemaphoreType.DMA,
        ],
    )(h, col2, row2, zeros_nd)


def _mm1_body(x_ref, w_ref, b_ref, dg_ref, o_ref, dis_ref):
    i = pl.program_id(0)
    deg = dg_ref[0][:, 0:1] + dg_ref[1][:, 0:1] + 1.0
    rows = i * BR + lax.broadcasted_iota(jnp.int32, (BR, 1), 0)
    dis = jnp.where(rows < N, lax.rsqrt(deg), 0.0)
    disf = jnp.broadcast_to(dis, (BR, D))
    h = jnp.dot(x_ref[...], w_ref[...],
                preferred_element_type=jnp.float32,
                precision=lax.Precision.HIGHEST) + b_ref[...]
    o_ref[...] = disf * h
    dis_ref[...] = disf


@jax.jit
def _tc_layer1(x_pad, w1t, b1r, degp):
    return pl.pallas_call(
        _mm1_body,
        grid=(NBLK,),
        in_specs=[
            pl.BlockSpec((BR, D), lambda i: (i, 0)),
            pl.BlockSpec((D, D), lambda i: (0, 0)),
            pl.BlockSpec((1, D), lambda i: (0, 0)),
            pl.BlockSpec((2, BR, 16), lambda i: (0, i, 0)),
        ],
        out_specs=[
            pl.BlockSpec((BR, D), lambda i: (i, 0)),
            pl.BlockSpec((BR, D), lambda i: (i, 0)),
        ],
        out_shape=[
            jax.ShapeDtypeStruct((NP, D), jnp.float32),
            jax.ShapeDtypeStruct((NP, D), jnp.float32),
        ],
        compiler_params=pltpu.CompilerParams(
            dimension_semantics=("parallel",)),
    )(x_pad, w1t, b1r, degp)


def _mm2_body(r_ref, h1_ref, dis_ref, w_ref, b_ref, o_ref):
    t = dis_ref[...] * (r_ref[0] + r_ref[1] + h1_ref[...])
    t = jnp.maximum(t, 0.0)
    h = jnp.dot(t, w_ref[...],
                preferred_element_type=jnp.float32,
                precision=lax.Precision.HIGHEST) + b_ref[...]
    o_ref[...] = dis_ref[...] * h


@jax.jit
def _tc_layer2(raw1, h1p, disf, w2t, b2r):
    return pl.pallas_call(
        _mm2_body,
        grid=(NBLK,),
        in_specs=[
            pl.BlockSpec((2, BR, D), lambda i: (0, i, 0)),
            pl.BlockSpec((BR, D), lambda i: (i, 0)),
            pl.BlockSpec((BR, D), lambda i: (i, 0)),
            pl.BlockSpec((D, D), lambda i: (0, 0)),
            pl.BlockSpec((1, D), lambda i: (0, 0)),
        ],
        out_specs=pl.BlockSpec((BR, D), lambda i: (i, 0)),
        out_shape=jax.ShapeDtypeStruct((NP, D), jnp.float32),
        compiler_params=pltpu.CompilerParams(
            dimension_semantics=("parallel",)),
    )(raw1, h1p, disf, w2t, b2r)


def _pool_body(r_ref, h2_ref, dis_ref, b_ref, o_ref, acc, cnt):
    i = pl.program_id(0)

    @pl.when(i == 0)
    def _():
        acc[...] = jnp.zeros((NG, D), jnp.float32)
        cnt[...] = jnp.zeros((NG, D), jnp.float32)

    h2 = dis_ref[...] * (r_ref[0] + r_ref[1] + h2_ref[...])
    onehot = (b_ref[...] == lax.broadcasted_iota(jnp.int32, (BR, NG), 1))
    onehot = onehot.astype(jnp.float32)
    dn = (((0,), (0,)), ((), ()))
    acc[...] += lax.dot_general(onehot, h2, dn,
                                preferred_element_type=jnp.float32,
                                precision=lax.Precision.HIGHEST)
    cnt[...] += lax.dot_general(onehot, jnp.ones((BR, D), jnp.float32), dn,
                                preferred_element_type=jnp.float32,
                                precision=lax.Precision.HIGHEST)

    @pl.when(i == NBLK - 1)
    def _():
        o_ref[...] = acc[...] / jnp.maximum(cnt[...], 1.0)


@jax.jit
def _tc_pool(raw2, h2p, disf, batch2d):
    return pl.pallas_call(
        _pool_body,
        grid=(NBLK,),
        in_specs=[
            pl.BlockSpec((2, BR, D), lambda i: (0, i, 0)),
            pl.BlockSpec((BR, D), lambda i: (i, 0)),
            pl.BlockSpec((BR, D), lambda i: (i, 0)),
            pl.BlockSpec((BR, 1), lambda i: (i, 0)),
        ],
        out_specs=pl.BlockSpec((NG, D), lambda i: (0, 0)),
        out_shape=jax.ShapeDtypeStruct((NG, D), jnp.float32),
        scratch_shapes=[
            pltpu.VMEM((NG, D), jnp.float32),
            pltpu.VMEM((NG, D), jnp.float32),
        ],
    )(raw2, h2p, disf, batch2d)


def kernel(x, edge_index, batch, W1, b1, W2, b2):
    row = edge_index[0]
    col = edge_index[1]
    pad = jnp.full((EP - E,), N, jnp.int32)
    col2 = jnp.concatenate([col, pad]).reshape(EP // CH, CH)
    row2 = jnp.concatenate([row, pad]).reshape(EP // CH, CH)
    x_pad = jnp.zeros((NP, D), jnp.float32).at[:N].set(x)
    batch2d = jnp.concatenate(
        [batch, jnp.full((NP - N,), NG, jnp.int32)]).reshape(NP, 1)
    zeros16 = jnp.zeros((NP, 16), jnp.float32)
    ones16 = jnp.ones((CH, 16), jnp.float32)
    zeros_nd = jnp.zeros((NP, D), jnp.float32)
    w1t = W1.T
    w2t = W2.T
    b1r = b1.reshape(1, D)
    b2r = b2.reshape(1, D)

    degp = _sc_degrees(col2, ones16, zeros16)
    h1p, disf = _tc_layer1(x_pad, w1t, b1r, degp)
    raw1 = _sc_aggregate(h1p, col2, row2, zeros_nd)
    h2p = _tc_layer2(raw1, h1p, disf, w2t, b2r)
    raw2 = _sc_aggregate(h2p, col2, row2, zeros_nd)
    return _tc_pool(raw2, h2p, disf, batch2d)


# speedup vs baseline: 9.3787x; 9.3787x over previous
import functools

import jax
import jax.numpy as jnp
from jax import lax
from jax.experimental import pallas as pl
from jax.experimental.pallas import tpu as pltpu
from jax.experimental.pallas import tpu_sc as plsc

N = 10000
E = 320000
D = 128
NG = 64
NP = 10240
EP = 323584
CH = 128
CPT = EP // (32 * CH)
RPS = NP // 16
NBLK = 16
BR = NP // NBLK

@functools.cache
def _sc_mesh():
    return plsc.VectorSubcoreMesh(core_axis_name="c", subcore_axis_name="s")


def _deg_body(col2_hbm, ones_hbm, zeros_hbm, out_hbm, acc_sh, idx_v, ones_v):
    c = lax.axis_index("c")
    s = lax.axis_index("s")
    pltpu.sync_copy(zeros_hbm.at[pl.ds(s * RPS, RPS)],
                    acc_sh.at[pl.ds(s * RPS, RPS)])
    pltpu.sync_copy(ones_hbm, ones_v)
    plsc.subcore_barrier()
    base = (c * 16 + s) * CPT

    @pl.loop(0, CPT)
    def _(j):
        pltpu.sync_copy(col2_hbm.at[base + j], idx_v.at[0])
        pltpu.sync_copy(ones_v, acc_sh.at[idx_v.at[0]], add=True)

    plsc.subcore_barrier()
    pltpu.sync_copy(acc_sh.at[pl.ds(s * RPS, RPS)],
                    out_hbm.at[c, pl.ds(s * RPS, RPS)])


@jax.jit
def _sc_degrees(col2, ones_nd, zeros_nd):
    return pl.kernel(
        _deg_body,
        out_type=jax.ShapeDtypeStruct((2, NP, D), jnp.float32),
        mesh=_sc_mesh(),
        scratch_types=[
            pltpu.VMEM_SHARED((NP, D), jnp.float32),
            pltpu.VMEM((1, CH), jnp.int32),
            pltpu.VMEM((CH, D), jnp.float32),
        ],
    )(col2, ones_nd, zeros_nd)


def _agg_body(h_hbm, col2_hbm, row2_hbm, zeros_hbm, out_hbm,
              acc_sh, cidx, ridx, msg, sem):
    c = lax.axis_index("c")
    s = lax.axis_index("s")
    pltpu.sync_copy(zeros_hbm.at[pl.ds(s * RPS, RPS)],
                    acc_sh.at[pl.ds(s * RPS, RPS)])
    plsc.subcore_barrier()
    base = (c * 16 + s) * CPT

    @pl.loop(0, CPT)
    def _(j):
        pltpu.sync_copy(col2_hbm.at[base + j], cidx.at[0])
        pltpu.sync_copy(row2_hbm.at[base + j], ridx.at[0])
        pltpu.async_copy(h_hbm.at[cidx.at[0]], msg, sem).wait()
        pltpu.sync_copy(msg, acc_sh.at[ridx.at[0]], add=True)

    plsc.subcore_barrier()
    pltpu.sync_copy(acc_sh.at[pl.ds(s * RPS, RPS)],
                    out_hbm.at[c, pl.ds(s * RPS, RPS)])


@jax.jit
def _sc_aggregate(h, col2, row2, zeros_nd):
    return pl.kernel(
        _agg_body,
        out_type=jax.ShapeDtypeStruct((2, NP, D), jnp.float32),
        mesh=_sc_mesh(),
        scratch_types=[
            pltpu.VMEM_SHARED((NP, D), jnp.float32),
            pltpu.VMEM((1, CH), jnp.int32),
            pltpu.VMEM((1, CH), jnp.int32),
            pltpu.VMEM((CH, D), jnp.float32),
            pltpu.SemaphoreType.DMA,
        ],
    )(h, col2, row2, zeros_nd)


def _mm1_body(x_ref, w_ref, b_ref, dg_ref, o_ref, dis_ref):
    i = pl.program_id(0)
    deg = dg_ref[0][:, 0:1] + dg_ref[1][:, 0:1] + 1.0
    rows = i * BR + lax.broadcasted_iota(jnp.int32, (BR, 1), 0)
    dis = jnp.where(rows < N, lax.rsqrt(deg), 0.0)
    disf = jnp.broadcast_to(dis, (BR, D))
    h = jnp.dot(x_ref[...], w_ref[...],
                preferred_element_type=jnp.float32,
                precision=lax.Precision.HIGHEST) + b_ref[...]
    o_ref[...] = disf * h
    dis_ref[...] = disf


@jax.jit
def _tc_layer1(x_pad, w1t, b1r, degp):
    return pl.pallas_call(
        _mm1_body,
        grid=(NBLK,),
        in_specs=[
            pl.BlockSpec((BR, D), lambda i: (i, 0)),
            pl.BlockSpec((D, D), lambda i: (0, 0)),
            pl.BlockSpec((1, D), lambda i: (0, 0)),
            pl.BlockSpec((2, BR, D), lambda i: (0, i, 0)),
        ],
        out_specs=[
            pl.BlockSpec((BR, D), lambda i: (i, 0)),
            pl.BlockSpec((BR, D), lambda i: (i, 0)),
        ],
        out_shape=[
            jax.ShapeDtypeStruct((NP, D), jnp.float32),
            jax.ShapeDtypeStruct((NP, D), jnp.float32),
        ],
        compiler_params=pltpu.CompilerParams(
            dimension_semantics=("parallel",)),
    )(x_pad, w1t, b1r, degp)


def _mm2_body(r_ref, h1_ref, dis_ref, w_ref, b_ref, o_ref):
    t = dis_ref[...] * (r_ref[0] + r_ref[1] + h1_ref[...])
    t = jnp.maximum(t, 0.0)
    h = jnp.dot(t, w_ref[...],
                preferred_element_type=jnp.float32,
                precision=lax.Precision.HIGHEST) + b_ref[...]
    o_ref[...] = dis_ref[...] * h


@jax.jit
def _tc_layer2(raw1, h1p, disf, w2t, b2r):
    return pl.pallas_call(
        _mm2_body,
        grid=(NBLK,),
        in_specs=[
            pl.BlockSpec((2, BR, D), lambda i: (0, i, 0)),
            pl.BlockSpec((BR, D), lambda i: (i, 0)),
            pl.BlockSpec((BR, D), lambda i: (i, 0)),
            pl.BlockSpec((D, D), lambda i: (0, 0)),
            pl.BlockSpec((1, D), lambda i: (0, 0)),
        ],
        out_specs=pl.BlockSpec((BR, D), lambda i: (i, 0)),
        out_shape=jax.ShapeDtypeStruct((NP, D), jnp.float32),
        compiler_params=pltpu.CompilerParams(
            dimension_semantics=("parallel",)),
    )(raw1, h1p, disf, w2t, b2r)


def _pool_body(r_ref, h2_ref, dis_ref, b_ref, o_ref, acc, cnt):
    i = pl.program_id(0)

    @pl.when(i == 0)
    def _():
        acc[...] = jnp.zeros((NG, D), jnp.float32)
        cnt[...] = jnp.zeros((NG, D), jnp.float32)

    h2 = dis_ref[...] * (r_ref[0] + r_ref[1] + h2_ref[...])
    onehot = (b_ref[...] == lax.broadcasted_iota(jnp.int32, (BR, NG), 1))
    onehot = onehot.astype(jnp.float32)
    dn = (((0,), (0,)), ((), ()))
    acc[...] += lax.dot_general(onehot, h2, dn,
                                preferred_element_type=jnp.float32,
                                precision=lax.Precision.HIGHEST)
    cnt[...] += lax.dot_general(onehot, jnp.ones((BR, D), jnp.float32), dn,
                                preferred_element_type=jnp.float32,
                                precision=lax.Precision.HIGHEST)

    @pl.when(i == NBLK - 1)
    def _():
        o_ref[...] = acc[...] / jnp.maximum(cnt[...], 1.0)


@jax.jit
def _tc_pool(raw2, h2p, disf, batch2d):
    return pl.pallas_call(
        _pool_body,
        grid=(NBLK,),
        in_specs=[
            pl.BlockSpec((2, BR, D), lambda i: (0, i, 0)),
            pl.BlockSpec((BR, D), lambda i: (i, 0)),
            pl.BlockSpec((BR, D), lambda i: (i, 0)),
            pl.BlockSpec((BR, 1), lambda i: (i, 0)),
        ],
        out_specs=pl.BlockSpec((NG, D), lambda i: (0, 0)),
        out_shape=jax.ShapeDtypeStruct((NG, D), jnp.float32),
        scratch_shapes=[
            pltpu.VMEM((NG, D), jnp.float32),
            pltpu.VMEM((NG, D), jnp.float32),
        ],
    )(raw2, h2p, disf, batch2d)


def kernel(x, edge_index, batch, W1, b1, W2, b2):
    row = edge_index[0]
    col = edge_index[1]
    pad = jnp.full((EP - E,), N, jnp.int32)
    col2 = jnp.concatenate([col, pad]).reshape(EP // CH, CH)
    row2 = jnp.concatenate([row, pad]).reshape(EP // CH, CH)
    x_pad = jnp.zeros((NP, D), jnp.float32).at[:N].set(x)
    batch2d = jnp.concatenate(
        [batch, jnp.full((NP - N,), NG, jnp.int32)]).reshape(NP, 1)
    zeros_nd = jnp.zeros((NP, D), jnp.float32)
    ones_nd = jnp.ones((CH, D), jnp.float32)
    w1t = W1.T
    w2t = W2.T
    b1r = b1.reshape(1, D)
    b2r = b2.reshape(1, D)

    degp = _sc_degrees(col2, ones_nd, zeros_nd)
    h1p, disf = _tc_layer1(x_pad, w1t, b1r, degp)
    raw1 = _sc_aggregate(h1p, col2, row2, zeros_nd)
    h2p = _tc_layer2(raw1, h1p, disf, w2t, b2r)
    raw2 = _sc_aggregate(h2p, col2, row2, zeros_nd)
    return _tc_pool(raw2, h2p, disf, batch2d)
